# baseline (speedup 1.0000x reference)
import functools

import jax
import jax.numpy as jnp
from jax import lax
from jax.experimental import pallas as pl
from jax.experimental.pallas import tpu as pltpu
from jax.experimental.pallas import tpu_sc as plsc

B, L, D = 4096, 200, 128
NC, NS = 2, 16
NW = NC * NS
BPW = B // NW
NCHUNK = 1
CHUNK = L // NCHUNK
NLANE = 16
DV = D // NLANE


def _make_pool_body(bpw, nchunk, chunk, d):
    seq = nchunk * chunk
    dv = d // NLANE

    def body(x_hbm, table_hbm, out_hbm, idx_v, buf0, buf1, pooled_v, sem0, sem1):
        wid = lax.axis_index("s") * NC + lax.axis_index("c")
        base = wid * bpw
        pltpu.sync_copy(x_hbm.at[pl.ds(base, bpw)], idx_v)

        bufs = (buf0, buf1)
        sems = (sem0, sem1)

        def fire(r, buf, sem):
            for c in range(nchunk):
                pltpu.async_copy(
                    table_hbm.at[idx_v.at[r, c]],
                    buf.at[pl.ds(c * chunk, chunk)],
                    sem,
                )

        def wait(buf, sem):
            pltpu.make_async_copy(table_hbm.at[pl.ds(0, seq)], buf, sem).wait()

        def reduce_row(r, buf):
            def step(i, accs):
                return tuple(
                    jnp.maximum(a, buf[i, pl.ds(NLANE * k, NLANE)])
                    for k, a in enumerate(accs)
                )
            init = tuple(buf[0, pl.ds(NLANE * k, NLANE)] for k in range(dv))
            accs = lax.fori_loop(1, seq, step, init, unroll=4)
            for k in range(dv):
                pooled_v[r, pl.ds(NLANE * k, NLANE)] = accs[k]

        fire(0, bufs[0], sems[0])

        def outer(g, _):
            for s in range(2):
                r = 2 * g + s
                @pl.when(r + 1 < bpw)
                def _fire_next():
                    fire(r + 1, bufs[1 - s], sems[1 - s])
                wait(bufs[s], sems[s])
                reduce_row(r, bufs[s])
            return 0

        lax.fori_loop(0, bpw // 2, outer, 0)
        pltpu.sync_copy(pooled_v, out_hbm.at[pl.ds(base, bpw)])

    return body


def _make_pool(bsz, bpw, nchunk, chunk, d, interpret=False):
    mesh = plsc.VectorSubcoreMesh(
        core_axis_name="c", subcore_axis_name="s", num_cores=NC, num_subcores=NS
    )
    seq = nchunk * chunk
    return pl.kernel(
        _make_pool_body(bpw, nchunk, chunk, d),
        out_type=jax.ShapeDtypeStruct((bsz, d), jnp.float32),
        mesh=mesh,
        scratch_types=[
            pltpu.VMEM((bpw, nchunk, chunk), jnp.int32),
            pltpu.VMEM((seq, d), jnp.float32),
            pltpu.VMEM((seq, d), jnp.float32),
            pltpu.VMEM((bpw, d), jnp.float32),
            pltpu.SemaphoreType.DMA,
            pltpu.SemaphoreType.DMA,
        ],
        interpret=interpret,
    )


def _mm_body(p_ref, w_ref, b_ref, o_ref):
    o_ref[...] = (
        jnp.dot(p_ref[...], w_ref[...], preferred_element_type=jnp.float32)
        + b_ref[...]
    )


_BM = 512


@jax.jit
def kernel(x, table, W, b):
    xi = x.astype(jnp.int32).reshape(B, NCHUNK, CHUNK)
    pooled = _make_pool(B, BPW, NCHUNK, CHUNK, D)(xi, table)
    out = pl.pallas_call(
        _mm_body,
        grid=(B // _BM,),
        in_specs=[
            pl.BlockSpec((_BM, D), lambda i: (i, 0)),
            pl.BlockSpec((D, D), lambda i: (0, 0)),
            pl.BlockSpec((1, D), lambda i: (0, 0)),
        ],
        out_specs=pl.BlockSpec((_BM, D), lambda i: (i, 0)),
        out_shape=jax.ShapeDtypeStruct((B, D), jnp.float32),
    )(pooled, W, b.reshape(1, D))
    return out


# speedup vs baseline: 1.2480x; 1.2480x over previous
import functools

import jax
import jax.numpy as jnp
from jax import lax
from jax.experimental import pallas as pl
from jax.experimental.pallas import tpu as pltpu
from jax.experimental.pallas import tpu_sc as plsc

B, L, D = 4096, 200, 128
NC, NS = 2, 16
NW = NC * NS
BPW = B // NW
NCHUNK = 2
NBUF = 3
CHUNK = L // NCHUNK
NLANE = 16
DV = D // NLANE


def _make_pool_body(bpw, nchunk, chunk, d):
    seq = nchunk * chunk
    dv = d // NLANE

    def body(x_hbm, table_hbm, out_hbm, idx_v, buf0, buf1, buf2, pooled_v,
             sem0, sem1, sem2):
        wid = lax.axis_index("s") * NC + lax.axis_index("c")
        base = wid * bpw
        pltpu.sync_copy(x_hbm.at[pl.ds(base, bpw)], idx_v)

        bufs = (buf0, buf1, buf2)
        sems = (sem0, sem1, sem2)

        def fire(r, buf, sem):
            for c in range(nchunk):
                pltpu.async_copy(
                    table_hbm.at[idx_v.at[r, c]],
                    buf.at[pl.ds(c * chunk, chunk)],
                    sem,
                )

        def wait(buf, sem):
            pltpu.make_async_copy(table_hbm.at[pl.ds(0, seq)], buf, sem).wait()

        def reduce_row(r, buf):
            def step(i, accs):
                return tuple(
                    jnp.maximum(a, buf[i, pl.ds(NLANE * k, NLANE)])
                    for k, a in enumerate(accs)
                )
            init = tuple(buf[0, pl.ds(NLANE * k, NLANE)] for k in range(dv))
            accs = lax.fori_loop(1, seq, step, init, unroll=4)
            for k in range(dv):
                pooled_v[r, pl.ds(NLANE * k, NLANE)] = accs[k]

        fire(0, bufs[0], sems[0])
        fire(1, bufs[1], sems[1])

        def handle(r, s):
            @pl.when(r + 2 < bpw)
            def _fire_next():
                fire(r + 2, bufs[(s + 2) % 3], sems[(s + 2) % 3])
            wait(bufs[s], sems[s])
            reduce_row(r, bufs[s])

        def outer(g, _):
            for s in range(3):
                handle(3 * g + s, s)
            return 0

        ngroups = (bpw - 2) // 3
        lax.fori_loop(0, ngroups, outer, 0)
        for r in range(3 * ngroups, bpw):
            handle(r, r % 3)
        pltpu.sync_copy(pooled_v, out_hbm.at[pl.ds(base, bpw)])

    return body


def _make_pool(bsz, bpw, nchunk, chunk, d, interpret=False):
    mesh = plsc.VectorSubcoreMesh(
        core_axis_name="c", subcore_axis_name="s", num_cores=NC, num_subcores=NS
    )
    seq = nchunk * chunk
    return pl.kernel(
        _make_pool_body(bpw, nchunk, chunk, d),
        out_type=jax.ShapeDtypeStruct((bsz, d), jnp.float32),
        mesh=mesh,
        scratch_types=[
            pltpu.VMEM((bpw, nchunk, chunk), jnp.int32),
            pltpu.VMEM((seq, d), jnp.float32),
            pltpu.VMEM((seq, d), jnp.float32),
            pltpu.VMEM((seq, d), jnp.float32),
            pltpu.VMEM((bpw, d), jnp.float32),
            pltpu.SemaphoreType.DMA,
            pltpu.SemaphoreType.DMA,
            pltpu.SemaphoreType.DMA,
        ],
        interpret=interpret,
    )


def _mm_body(p_ref, w_ref, b_ref, o_ref):
    o_ref[...] = (
        jnp.dot(p_ref[...], w_ref[...], preferred_element_type=jnp.float32)
        + b_ref[...]
    )


_BM = 512


@jax.jit
def kernel(x, table, W, b):
    xi = x.astype(jnp.int32).reshape(B, NCHUNK, CHUNK)
    pooled = _make_pool(B, BPW, NCHUNK, CHUNK, D)(xi, table)
    out = pl.pallas_call(
        _mm_body,
        grid=(B // _BM,),
        in_specs=[
            pl.BlockSpec((_BM, D), lambda i: (i, 0)),
            pl.BlockSpec((D, D), lambda i: (0, 0)),
            pl.BlockSpec((1, D), lambda i: (0, 0)),
        ],
        out_specs=pl.BlockSpec((_BM, D), lambda i: (i, 0)),
        out_shape=jax.ShapeDtypeStruct((B, D), jnp.float32),
    )(pooled, W, b.reshape(1, D))
    return out


# speedup vs baseline: 1.2533x; 1.0042x over previous
import functools

import jax
import jax.numpy as jnp
from jax import lax
from jax.experimental import pallas as pl
from jax.experimental.pallas import tpu as pltpu
from jax.experimental.pallas import tpu_sc as plsc

B, L, D = 4096, 200, 128
NC, NS = 2, 16
NW = NC * NS
BPW = B // NW
NCHUNK = 2
CHUNK = L // NCHUNK
NBUF = 6
NLANE = 16
DV = D // NLANE


def _make_pool_body(bpw, chunk, d):
    dv = d // NLANE
    nhalf = bpw * NCHUNK

    def body(x_hbm, table_hbm, out_hbm, idx_v, b0, b1, b2, b3, b4, b5,
             pooled_v, s0, s1, s2, s3, s4, s5):
        wid = lax.axis_index("s") * NC + lax.axis_index("c")
        base = wid * bpw
        pltpu.sync_copy(x_hbm.at[pl.ds(base, bpw)], idx_v)

        bufs = (b0, b1, b2, b3, b4, b5)
        sems = (s0, s1, s2, s3, s4, s5)

        def fire(r, c, buf, sem):
            pltpu.async_copy(table_hbm.at[idx_v.at[r, c]], buf, sem)

        def wait(r, c, buf, sem):
            pltpu.make_async_copy(table_hbm.at[idx_v.at[r, c]], buf, sem).wait()

        def reduce_half(buf, accs):
            lo = 0
            if accs is None:
                accs = tuple(buf[0, pl.ds(NLANE * k, NLANE)] for k in range(dv))
                lo = 1

            def step(i, a):
                return tuple(
                    jnp.maximum(v, buf[i, pl.ds(NLANE * k, NLANE)])
                    for k, v in enumerate(a)
                )
            return lax.fori_loop(lo, chunk, step, accs, unroll=4)

        for h in range(NBUF - 1):
            fire(h // NCHUNK, h % NCHUNK, bufs[h], sems[h])

        def outer(g, _):
            accs = None
            for s in range(NBUF):
                r_cur = NBUF // NCHUNK * g + s // NCHUNK
                r_fire = NBUF // NCHUNK * g + (s + NBUF - 1) // NCHUNK
                sel = (s + NBUF - 1) % NBUF
                fire(r_fire, (s + NBUF - 1) % NCHUNK, bufs[sel], sems[sel])
                wait(r_cur, s % NCHUNK, bufs[s], sems[s])
                accs = reduce_half(bufs[s], accs)
                if s % NCHUNK == NCHUNK - 1:
                    for k in range(dv):
                        pooled_v[r_cur, pl.ds(NLANE * k, NLANE)] = accs[k]
                    accs = None
            return 0

        ngroups = (nhalf - (NBUF - 1)) // NBUF
        lax.fori_loop(0, ngroups, outer, 0)
        accs = None
        for h in range(NBUF * ngroups, nhalf):
            s = h % NBUF
            if h + NBUF - 1 < nhalf:
                fire((h + NBUF - 1) // NCHUNK, (h + NBUF - 1) % NCHUNK,
                     bufs[(s + NBUF - 1) % NBUF], sems[(s + NBUF - 1) % NBUF])
            wait(h // NCHUNK, h % NCHUNK, bufs[s], sems[s])
            accs = reduce_half(bufs[s], accs)
            if h % NCHUNK == NCHUNK - 1:
                for k in range(dv):
                    pooled_v[h // NCHUNK, pl.ds(NLANE * k, NLANE)] = accs[k]
                accs = None

        pltpu.sync_copy(pooled_v, out_hbm.at[pl.ds(base, bpw)])

    return body


def _make_pool(bsz, bpw, chunk, d, interpret=False):
    mesh = plsc.VectorSubcoreMesh(
        core_axis_name="c", subcore_axis_name="s", num_cores=NC, num_subcores=NS
    )
    return pl.kernel(
        _make_pool_body(bpw, chunk, d),
        out_type=jax.ShapeDtypeStruct((bsz, d), jnp.float32),
        mesh=mesh,
        scratch_types=[
            pltpu.VMEM((bpw, NCHUNK, chunk), jnp.int32),
        ]
        + [pltpu.VMEM((chunk, d), jnp.float32) for _ in range(NBUF)]
        + [pltpu.VMEM((bpw, d), jnp.float32)]
        + [pltpu.SemaphoreType.DMA for _ in range(NBUF)],
        interpret=interpret,
    )


def _mm_body(p_ref, w_ref, b_ref, o_ref):
    o_ref[...] = (
        jnp.dot(p_ref[...], w_ref[...], preferred_element_type=jnp.float32)
        + b_ref[...]
    )


_BM = 512


@jax.jit
def kernel(x, table, W, b):
    xi = x.astype(jnp.int32).reshape(B, NCHUNK, CHUNK)
    pooled = _make_pool(B, BPW, CHUNK, D)(xi, table)
    out = pl.pallas_call(
        _mm_body,
        grid=(B // _BM,),
        in_specs=[
            pl.BlockSpec((_BM, D), lambda i: (i, 0)),
            pl.BlockSpec((D, D), lambda i: (0, 0)),
            pl.BlockSpec((1, D), lambda i: (0, 0)),
        ],
        out_specs=pl.BlockSpec((_BM, D), lambda i: (i, 0)),
        out_shape=jax.ShapeDtypeStruct((B, D), jnp.float32),
    )(pooled, W, b.reshape(1, D))
    return out


# speedup vs baseline: 1.2561x; 1.0022x over previous
import jax
import jax.numpy as jnp
from jax import lax
from jax.experimental import pallas as pl
from jax.experimental.pallas import tpu as pltpu
from jax.experimental.pallas import tpu_sc as plsc

B, L, D = 4096, 200, 128
NC, NS = 2, 16
NW = NC * NS
BPW = B // NW
NCHUNK = 2
CHUNK = L // NCHUNK
NBUF = 6
NLANE = 16


def _make_pool_body(bpw, chunk, d):
    dv = d // NLANE
    nhalf = bpw * NCHUNK

    def body(x_hbm, table_hbm, out_hbm, idx_v, b0, b1, b2, b3, b4, b5,
             pooled_v, s0, s1, s2, s3, s4, s5):
        wid = lax.axis_index("s") * NC + lax.axis_index("c")
        base = wid * bpw
        pltpu.sync_copy(x_hbm.at[pl.ds(base, bpw)], idx_v)

        bufs = (b0, b1, b2, b3, b4, b5)
        sems = (s0, s1, s2, s3, s4, s5)

        def fire(r, c, buf, sem):
            pltpu.async_copy(table_hbm.at[idx_v.at[r, c]], buf, sem)

        def wait(r, c, buf, sem):
            pltpu.make_async_copy(table_hbm.at[idx_v.at[r, c]], buf, sem).wait()

        def reduce_half(buf, accs):
            lo = 0
            if accs is None:
                accs = tuple(buf[0, pl.ds(NLANE * k, NLANE)] for k in range(dv))
                lo = 1

            def step(i, a):
                return tuple(
                    jnp.maximum(v, buf[i, pl.ds(NLANE * k, NLANE)])
                    for k, v in enumerate(a)
                )
            return lax.fori_loop(lo, chunk, step, accs, unroll=4)

        for h in range(NBUF - 1):
            fire(h // NCHUNK, h % NCHUNK, bufs[h], sems[h])

        def outer(g, _):
            accs = None
            for s in range(NBUF):
                r_cur = NBUF // NCHUNK * g + s // NCHUNK
                r_fire = NBUF // NCHUNK * g + (s + NBUF - 1) // NCHUNK
                sel = (s + NBUF - 1) % NBUF
                fire(r_fire, (s + NBUF - 1) % NCHUNK, bufs[sel], sems[sel])
                wait(r_cur, s % NCHUNK, bufs[s], sems[s])
                accs = reduce_half(bufs[s], accs)
                if s % NCHUNK == NCHUNK - 1:
                    for k in range(dv):
                        pooled_v[r_cur, pl.ds(NLANE * k, NLANE)] = accs[k]
                    accs = None
            return 0

        ngroups = (nhalf - (NBUF - 1)) // NBUF
        lax.fori_loop(0, ngroups, outer, 0)
        accs = None
        for h in range(NBUF * ngroups, nhalf):
            s = h % NBUF
            if h + NBUF - 1 < nhalf:
                fire((h + NBUF - 1) // NCHUNK, (h + NBUF - 1) % NCHUNK,
                     bufs[(s + NBUF - 1) % NBUF], sems[(s + NBUF - 1) % NBUF])
            wait(h // NCHUNK, h % NCHUNK, bufs[s], sems[s])
            accs = reduce_half(bufs[s], accs)
            if h % NCHUNK == NCHUNK - 1:
                for k in range(dv):
                    pooled_v[h // NCHUNK, pl.ds(NLANE * k, NLANE)] = accs[k]
                accs = None

        pltpu.sync_copy(pooled_v, out_hbm.at[pl.ds(base, bpw)])

    return body


def _make_pool(bsz, bpw, chunk, d, interpret=False):
    mesh = plsc.VectorSubcoreMesh(
        core_axis_name="c", subcore_axis_name="s", num_cores=NC, num_subcores=NS
    )
    return pl.kernel(
        _make_pool_body(bpw, chunk, d),
        out_type=jax.ShapeDtypeStruct((bsz, d), jnp.float32),
        mesh=mesh,
        scratch_types=[
            pltpu.VMEM((bpw, NCHUNK, chunk), jnp.int32),
        ]
        + [pltpu.VMEM((chunk, d), jnp.float32) for _ in range(NBUF)]
        + [pltpu.VMEM((bpw, d), jnp.float32)]
        + [pltpu.SemaphoreType.DMA for _ in range(NBUF)],
        interpret=interpret,
    )


def _mm_body(p_ref, w_ref, b_ref, o_ref):
    o_ref[...] = (
        jnp.dot(p_ref[...], w_ref[...], preferred_element_type=jnp.float32)
        + b_ref[...]
    )


_BM = 512


@jax.jit
def kernel(x, table, W, b):
    xi = x.astype(jnp.int32).reshape(B, NCHUNK, CHUNK)
    pooled = _make_pool(B, BPW, CHUNK, D)(xi, table)
    out = pl.pallas_call(
        _mm_body,
        grid=(B // _BM,),
        in_specs=[
            pl.BlockSpec((_BM, D), lambda i: (i, 0)),
            pl.BlockSpec((D, D), lambda i: (0, 0)),
            pl.BlockSpec((1, D), lambda i: (0, 0)),
        ],
        out_specs=pl.BlockSpec((_BM, D), lambda i: (i, 0)),
        out_shape=jax.ShapeDtypeStruct((B, D), jnp.float32),
    )(pooled, W, b.reshape(1, D))
    return out
